# Initial kernel scaffold; baseline (speedup 1.0000x reference)
import jax
import jax.numpy as jnp
from jax.experimental import pallas as pl


def kernel(x, edge_index, batch, W0, b0, W1, b1, W2, b2, W3, b3, Wc1, bc1, Wc2, bc2, Wl, bl):
    raise NotImplementedError("write your pallas kernel here")


# speedup vs baseline: 1.0821x; 1.0821x over previous
import functools

import jax
import jax.numpy as jnp
from jax import lax
from jax.experimental import pallas as pl
from jax.experimental.pallas import tpu as pltpu
from jax.experimental.pallas import tpu_sc as plsc

N = 10000
E = 320000
G = 256
H = 128
K = 30
D = 385
DP = 512
OUT = 2

NR = 10112
ZROW = N
NW = 32
CHUNK = 128

DEGR = 10496
BOFF = 10112
BGARB = 10368
IDX2 = 344064
EPT2 = IDX2 // NW
NCHUNK2 = EPT2 // CHUNK
E2P_PAD = 331776 - (E + N)
BPAD = IDX2 - 331776 - N

GPW = G // NW


def _mesh():
    return plsc.VectorSubcoreMesh(core_axis_name="c", subcore_axis_name="s")


def _vperm(v, perm):
    return lax.gather(
        v, perm[:, None],
        lax.GatherDimensionNumbers(offset_dims=(), collapsed_slice_dims=(0,),
                                   start_index_map=(0,)),
        (1,), mode=lax.GatherScatterMode.PROMISE_IN_BOUNDS)


def _make_scatter_ones():
    rpt = DEGR // 16

    @functools.partial(
        pl.kernel,
        mesh=_mesh(),
        compiler_params=pltpu.CompilerParams(use_tc_tiling_on_sc=False),
        out_type=jax.ShapeDtypeStruct((2 * DEGR, 16), jnp.float32),
        scratch_types=[
            pltpu.VMEM((CHUNK,), jnp.int32),
            pltpu.VMEM((CHUNK, 16), jnp.float32),
            pltpu.VMEM_SHARED((DEGR, 16), jnp.float32),
        ],
    )
    def k(idx_hbm, zeros_hbm, ones_hbm, out_hbm, didx_v, ones_v, acc_sh):
        c = lax.axis_index("c")
        s = lax.axis_index("s")
        wid = s * 2 + c
        r0 = s * rpt
        pltpu.sync_copy(zeros_hbm.at[pl.ds(0, rpt), :],
                        acc_sh.at[pl.ds(r0, rpt), :])
        pltpu.sync_copy(ones_hbm, ones_v)
        plsc.subcore_barrier()

        base = wid * EPT2

        def body(i, carry):
            off = base + i * CHUNK
            pltpu.sync_copy(idx_hbm.at[pl.ds(off, CHUNK)], didx_v)
            pltpu.sync_copy(ones_v, acc_sh.at[didx_v], add=True)
            return carry

        lax.fori_loop(0, NCHUNK2, body, 0)
        plsc.subcore_barrier()
        pltpu.sync_copy(acc_sh.at[pl.ds(r0, rpt), :],
                        out_hbm.at[pl.ds(c * DEGR + r0, rpt), :])

    return k


def _make_topk():
    @functools.partial(
        pl.kernel,
        mesh=_mesh(),
        compiler_params=pltpu.CompilerParams(use_tc_tiling_on_sc=False),
        out_type=jax.ShapeDtypeStruct((G * 32, DP), jnp.float32),
        scratch_types=[
            pltpu.VMEM((NR,), jnp.float32),
            pltpu.VMEM((16,), jnp.int32),
            pltpu.VMEM((16,), jnp.int32),
            pltpu.VMEM((GPW * 32,), jnp.int32),
            pltpu.VMEM((32, DP), jnp.float32),
            pltpu.SemaphoreType.DMA,
        ],
    )
    def k(key_hbm, starts_hbm, counts_hbm, feat_hbm, out_hbm,
          key_v, st_v, cnt_v, idx_v, rows_v, sem):
        wid = lax.axis_index("s") * 2 + lax.axis_index("c")
        pltpu.sync_copy(key_hbm, key_v)
        pltpu.sync_copy(starts_hbm.at[pl.ds(wid * GPW, 16)], st_v)
        pltpu.sync_copy(counts_hbm.at[pl.ds(wid * GPW, 16)], cnt_v)
        lane = lax.broadcasted_iota(jnp.int32, (16,), 0)
        st16 = st_v[...]
        cnt16 = cnt_v[...]

        BIG = 4e9
        IBIG = 1 << 30
        SC1 = jnp.float32(2.0 ** 30)
        SC2 = jnp.float32(1e38)

        def ieq(a, b):
            return 1 - jnp.minimum(jnp.abs(a - b), 1)

        for jl in range(GPW):
            start = st16[jl]
            cnt = cnt16[jl]
            j0 = start // 16
            j1 = (start + cnt + 15) // 16
            end = start + cnt
            pos0 = jl * 32

            def masked_vals(j):
                v = key_v[pl.ds(j * 16, 16)]
                gl = j * 16 + lane
                mi = (jnp.minimum(jnp.maximum(gl - start + 1, 0), 1)
                      * jnp.minimum(jnp.maximum(end - gl, 0), 1))
                mf = mi.astype(jnp.float32)
                return v * mf + BIG * (1.0 - mf), gl

            def min_chunk(j, bv):
                vm, _ = masked_vals(j)
                return jnp.minimum(bv, vm)

            def select_k(kk, carry2):
                cur0, cur1 = carry2
                bv = lax.fori_loop(j0, j1, min_chunk,
                                   jnp.full((16,), BIG, jnp.float32))
                for sh in (8, 4, 2, 1):
                    bv = jnp.minimum(bv, _vperm(bv, (lane + sh) & 15))
                def idx_chunk(j, bi):
                    vm, gl = masked_vals(j)
                    d = vm - bv
                    dz = jnp.minimum(((d * SC1) * SC1) * SC2, 1.0)
                    ei = (1.0 - dz).astype(jnp.int32)
                    gi = gl * ei + IBIG * (1 - ei)
                    return jnp.minimum(bi, gi)

                bi = lax.fori_loop(j0, j1, idx_chunk,
                                   jnp.full((16,), IBIG, jnp.int32))
                for sh in (8, 4, 2, 1):
                    bi = jnp.minimum(bi, _vperm(bi, (lane + sh) & 15))
                idx_s = bi[0]
                vi = jnp.minimum(jnp.maximum(cnt - kk, 0), 1)
                u0 = ieq(lane, kk) * vi
                u1 = ieq(lane, kk - 16) * vi
                cur0 = cur0 * (1 - u0) + idx_s * u0
                cur1 = cur1 * (1 - u1) + idx_s * u1
                cbase = jnp.minimum((idx_s // 16) * 16, NR - 16)
                vcl = key_v[pl.ds(cbase, 16)]
                ef = (ieq(cbase + lane, idx_s) * vi).astype(jnp.float32)
                key_v[pl.ds(cbase, 16)] = vcl * (1.0 - ef) + BIG * ef
                return cur0, cur1

            zsplat = jnp.full((16,), ZROW, jnp.int32)
            cur0, cur1 = lax.fori_loop(0, K, select_k, (zsplat, zsplat))
            idx_v[pl.ds(pos0, 16)] = cur0
            idx_v[pl.ds(pos0 + 16, 16)] = cur1
            pltpu.async_copy(feat_hbm.at[idx_v.at[pl.ds(jl * 32, 32)]],
                             rows_v, sem).wait()
            g = wid * GPW + jl
            pltpu.sync_copy(rows_v, out_hbm.at[pl.ds(g * 32, 32), :])

    return k


_SC_CACHE = {}


def _cached(name, maker):
    if name not in _SC_CACHE:
        _SC_CACHE[name] = maker()
    return _SC_CACHE[name]


def _scatter_ones(*a):
    return _cached("ones", _make_scatter_ones)(*a)


def _topk(*a):
    return _cached("topk", _make_topk)(*a)


def _tc_mm(a, w):
    def body(a_ref, w_ref, o_ref):
        o_ref[...] = jnp.dot(a_ref[...], w_ref[...],
                             preferred_element_type=jnp.float32)

    return pl.pallas_call(
        body, out_shape=jax.ShapeDtypeStruct((a.shape[0], w.shape[1]),
                                             jnp.float32))(a, w)


def _tc_tanh_b(a, b):
    def body(a_ref, b_ref, o_ref):
        o_ref[...] = jnp.tanh(a_ref[...] + b_ref[...])

    return pl.pallas_call(
        body, out_shape=jax.ShapeDtypeStruct(a.shape, jnp.float32))(
            a, b.reshape(1, -1))


def _tc_head(pooled, wc1p, bc1, wc2r, bc2, wlp, bl):
    hp = lax.Precision.HIGHEST

    def body(po_ref, w1_ref, b1_ref, w2_ref, b2_ref, wl_ref, bl_ref, o_ref):
        a = po_ref[...]
        g1 = jnp.dot(a, w1_ref[...], preferred_element_type=jnp.float32,
                     precision=hp)
        g1 = jnp.maximum(g1 + b1_ref[...], 0.0)
        pair = g1.reshape(G * 15, 2, 16)
        zp = jnp.maximum(pair[:, 0, :], pair[:, 1, :])
        zp3 = zp.reshape(G, 15, 16)
        acc = jnp.zeros((G * 11, 32), jnp.float32)
        for j in range(5):
            win = zp3[:, j:j + 11, :].reshape(G * 11, 16)
            acc = acc + jnp.dot(win, w2_ref[j],
                                preferred_element_type=jnp.float32,
                                precision=hp)
        z2 = jnp.maximum(acc + b2_ref[...], 0.0)
        z4 = z2.reshape(G, 11, 32)
        o = bl_ref[...] + jnp.zeros((G, OUT), jnp.float32)
        for t in range(11):
            o = o + jnp.dot(z4[:, t, :], wl_ref[t],
                            preferred_element_type=jnp.float32, precision=hp)
        o_ref[...] = o

    return pl.pallas_call(
        body,
        out_shape=jax.ShapeDtypeStruct((G, OUT), jnp.float32),
    )(pooled, wc1p, bc1.reshape(1, 16), wc2r, bc2.reshape(1, 32), wlp,
      bl.reshape(1, OUT))


def kernel(x, edge_index, batch, W0, b0, W1, b1, W2, b2, W3, b3,
           Wc1, bc1, Wc2, bc2, Wl, bl):
    i32 = jnp.int32
    loop = jnp.arange(N, dtype=edge_index.dtype)
    src = jnp.concatenate([edge_index[0], loop])
    dst = jnp.concatenate([edge_index[1], loop])

    idx_deg = jnp.concatenate([dst.astype(i32),
                               jnp.full((E2P_PAD,), ZROW, i32),
                               batch.astype(i32) + BOFF,
                               jnp.full((BPAD,), BGARB, i32)])
    zeros_deg = jnp.zeros((DEGR // 16, 16), jnp.float32)
    ones16 = jnp.ones((CHUNK, 16), jnp.float32)
    degp = _scatter_ones(idx_deg, zeros_deg, ones16)
    degs = degp[:DEGR, 0] + degp[DEGR:, 0]
    deg = degs[:N]
    counts = degs[BOFF:BOFF + G].astype(i32)
    starts = jnp.cumsum(counts) - counts

    dinv = jnp.where(deg > 0, 1.0 / jnp.sqrt(deg), 0.0)
    norm = dinv[src] * dinv[dst]

    xs = []
    h = x
    for (W, b) in ((W0, b0), (W1, b1), (W2, b2), (W3, b3)):
        hw = _tc_mm(h, W)
        msg = hw[src] * norm[:, None]
        agg = jax.ops.segment_sum(msg, dst, num_segments=N)
        h = _tc_tanh_b(agg, b)
        xs.append(h)

    featr = jnp.concatenate(xs, axis=-1)
    key = batch.astype(jnp.float32) * 10.0 - featr[:, -1]

    feat = jnp.pad(featr, ((0, NR - N), (0, DP - D)))
    keyp = jnp.pad(key, (0, NR - N))
    starts_p = jnp.pad(starts.astype(i32), (0, 16), constant_values=N)
    counts_p = jnp.pad(counts, (0, 16))

    pooled32 = _topk(keyp, starts_p, counts_p, feat)
    pooled = pooled32.reshape(G, 32, DP)[:, :K, :].reshape(G * K, DP)

    wc1p = jnp.zeros((DP, 16), jnp.float32).at[:D].set(Wc1[:, 0, :].T)
    wc2r = Wc2.transpose(2, 1, 0)
    wlp = Wl.reshape(32, 11, OUT).transpose(1, 0, 2)

    return _tc_head(pooled, wc1p, bc1, wc2r, bc2, wlp, bl)
